# Initial kernel scaffold; baseline (speedup 1.0000x reference)
import jax
import jax.numpy as jnp
from jax.experimental import pallas as pl


def kernel(x, embed_table, W, b):
    raise NotImplementedError("write your pallas kernel here")


# speedup vs baseline: 4.8575x; 4.8575x over previous
import functools

import jax
import jax.numpy as jnp
from jax import lax
from jax.experimental import pallas as pl
from jax.experimental.pallas import tpu as pltpu
from jax.experimental.pallas import tpu_sc as plsc


def _fold_table(table, W, bias):
    def body(t_ref, w_ref, b_ref, o_ref):
        o_ref[...] = (
            lax.dot_general(
                t_ref[...], w_ref[...],
                dimension_numbers=(((1,), (1,)), ((), ())),
                preferred_element_type=jnp.float32,
            )
            + b_ref[...]
        )

    return pl.pallas_call(
        body,
        out_shape=jax.ShapeDtypeStruct((table.shape[0], W.shape[0]), jnp.float32),
    )(table, W, bias.reshape(1, -1))


def _sc_gather(t_folded, idx_flat):
    n = idx_flat.shape[0]
    h = t_folded.shape[1]
    v = t_folded.shape[0]

    info = plsc.get_sparse_core_info()
    nw = info.num_cores * info.num_subcores
    per_w = n // nw
    chunk = 2048
    chunks = per_w // chunk
    unroll = 16

    idx3 = idx_flat.reshape(nw, chunks, chunk)
    mesh = plsc.VectorSubcoreMesh(core_axis_name="c", subcore_axis_name="s")

    @functools.partial(
        pl.kernel,
        mesh=mesh,
        compiler_params=pltpu.CompilerParams(use_tc_tiling_on_sc=False),
        out_type=jax.ShapeDtypeStruct((n, h), jnp.float32),
        scratch_types=[
            pltpu.VMEM((v, h), jnp.float32),
            pltpu.VMEM((chunk,), jnp.int32),
            pltpu.VMEM((chunk, h), jnp.float32),
        ],
    )
    def gather_kernel(t_hbm, idx_hbm, out_hbm, t_v, idx_v, rows_v):
        wid = lax.axis_index("s") * info.num_cores + lax.axis_index("c")
        pltpu.sync_copy(t_hbm, t_v)

        def chunk_body(c, carry):
            pltpu.sync_copy(idx_hbm.at[wid, c], idx_v)

            def row_body(r, carry2):
                base = r * unroll
                idx16 = idx_v[pl.ds(base, 16)]
                for u in range(unroll):
                    vi = idx16[u]
                    rows_v[base + u, pl.ds(0, 16)] = t_v[vi, pl.ds(0, 16)]
                    rows_v[base + u, pl.ds(16, 16)] = t_v[vi, pl.ds(16, 16)]
                return carry2

            lax.fori_loop(0, chunk // unroll, row_body, 0)
            row0 = wid * per_w + c * chunk
            pltpu.sync_copy(rows_v, out_hbm.at[pl.ds(row0, chunk)])
            return carry

        lax.fori_loop(0, chunks, chunk_body, 0)

    return gather_kernel(t_folded, idx3)


def kernel(x, embed_table, W, b):
    bsz, seqlen = x.shape
    hidden = embed_table.shape[1]
    t_folded = _fold_table(embed_table, W, b)
    idx_flat = x.reshape(-1).astype(jnp.int32)
    out_flat = _sc_gather(t_folded, idx_flat)
    return out_flat.reshape(bsz, seqlen, hidden)


# speedup vs baseline: 5.1777x; 1.0659x over previous
import functools

import jax
import jax.numpy as jnp
from jax import lax
from jax.experimental import pallas as pl
from jax.experimental.pallas import tpu as pltpu
from jax.experimental.pallas import tpu_sc as plsc


def _fold_table(table, W, bias):
    def body(t_ref, w_ref, b_ref, o_ref):
        o_ref[...] = (
            lax.dot_general(
                t_ref[...], w_ref[...],
                dimension_numbers=(((1,), (1,)), ((), ())),
                preferred_element_type=jnp.float32,
            )
            + b_ref[...]
        )

    return pl.pallas_call(
        body,
        out_shape=jax.ShapeDtypeStruct((table.shape[0], W.shape[0]), jnp.float32),
    )(table, W, bias.reshape(1, -1))


def _sc_gather(t_folded, x):
    bsz, seqlen = x.shape
    h = t_folded.shape[1]
    v = t_folded.shape[0]

    info = plsc.get_sparse_core_info()
    nw = info.num_cores * info.num_subcores
    per_w = bsz // nw
    cb = 8
    rows = cb * seqlen
    chunks = per_w // cb
    groups = rows // 16

    idx3 = x.reshape(nw, chunks, rows)
    mesh = plsc.VectorSubcoreMesh(core_axis_name="c", subcore_axis_name="s")

    @functools.partial(
        pl.kernel,
        mesh=mesh,
        compiler_params=pltpu.CompilerParams(use_tc_tiling_on_sc=False),
        out_type=jax.ShapeDtypeStruct((bsz, seqlen, h), jnp.float32),
        scratch_types=[
            pltpu.VMEM((v, h), jnp.float32),
            pltpu.VMEM((2, rows), jnp.int32),
            pltpu.VMEM((2, rows, h), jnp.float32),
            pltpu.SemaphoreType.DMA,
            pltpu.SemaphoreType.DMA,
        ],
    )
    def gather_kernel(t_hbm, idx_hbm, out_hbm, t_v, idx_v, rows_v, sem_in, sem_out):
        wid = lax.axis_index("s") * info.num_cores + lax.axis_index("c")
        b0w = wid * per_w
        pltpu.sync_copy(t_hbm, t_v)
        pltpu.async_copy(idx_hbm.at[wid, 0], idx_v.at[0], sem_in).wait()

        def do_chunk(c, p):
            @pl.when(c + 1 < chunks)
            def _():
                pltpu.async_copy(idx_hbm.at[wid, c + 1], idx_v.at[1 - p], sem_in)

            @pl.when(c >= 2)
            def _():
                for b in range(cb):
                    gb = b0w + (c - 2) * cb + b
                    pltpu.make_async_copy(
                        rows_v.at[p, pl.ds(b * seqlen, seqlen)],
                        out_hbm.at[gb],
                        sem_out,
                    ).wait()

            rv = rows_v.at[p]
            ixp = idx_v.at[p]

            def group_body(g, carry2):
                base = g * 16
                idx16 = ixp[pl.ds(base, 16)]
                for u in range(16):
                    vi = idx16[u]
                    rv[base + u, pl.ds(0, 16)] = t_v[vi, pl.ds(0, 16)]
                    rv[base + u, pl.ds(16, 16)] = t_v[vi, pl.ds(16, 16)]
                return carry2

            lax.fori_loop(0, groups, group_body, 0)

            for b in range(cb):
                gb = b0w + c * cb + b
                pltpu.async_copy(
                    rv.at[pl.ds(b * seqlen, seqlen)], out_hbm.at[gb], sem_out
                )

        def two_chunks(cc, carry):
            c = cc * 2

            @pl.when(c > 0)
            def _():
                pltpu.make_async_copy(
                    idx_hbm.at[wid, c], idx_v.at[0], sem_in
                ).wait()

            do_chunk(c, 0)

            pltpu.make_async_copy(
                idx_hbm.at[wid, c + 1], idx_v.at[1], sem_in
            ).wait()
            do_chunk(c + 1, 1)
            return carry

        lax.fori_loop(0, chunks // 2, two_chunks, 0)

        for c, p in ((chunks - 2, 0), (chunks - 1, 1)):
            for b in range(cb):
                gb = b0w + c * cb + b
                pltpu.make_async_copy(
                    rows_v.at[p, pl.ds(b * seqlen, seqlen)],
                    out_hbm.at[gb],
                    sem_out,
                ).wait()

    return gather_kernel(t_folded, idx3)


def kernel(x, embed_table, W, b):
    t_folded = _fold_table(embed_table, W, b)
    return _sc_gather(t_folded, x.astype(jnp.int32))
